# baseline (speedup 1.0000x reference)
import functools

import jax
import jax.numpy as jnp
from jax import lax
from jax.experimental import pallas as pl
from jax.experimental.pallas import tpu as pltpu
from jax.experimental.pallas import tpu_sc as plsc

N_NODES = 10000
IN_F = 128
OUT_F = 128
N_REL = 4
N_EDGES = 320000

NC = 2
NS = 16
CH = 80
NSLOT = 4
SUP = 5 * CH
_DEG_PAD = 40960


def _mm_body(x_ref, w_ref, y_ref):
    x = x_ref[...]
    for r in range(N_REL + 1):
        y_ref[r] = jnp.dot(x, w_ref[r], preferred_element_type=jnp.float32)


def _relation_matmul(x, w5):
    nb = 10
    bs = N_NODES // nb
    r = w5.shape[0]
    return pl.pallas_call(
        _mm_body,
        grid=(nb,),
        in_specs=[
            pl.BlockSpec((bs, IN_F), lambda i: (i, 0)),
            pl.BlockSpec((r, IN_F, OUT_F), lambda i: (0, 0, 0)),
        ],
        out_specs=pl.BlockSpec((r, bs, OUT_F), lambda i: (0, i, 0)),
        out_shape=jax.ShapeDtypeStruct((r, N_NODES, OUT_F), jnp.float32),
    )(x, w5)


def _fin_body(a_ref, y5_ref, b_ref, o_ref):
    o_ref[...] = jnp.maximum(
        a_ref[0] + a_ref[1] + y5_ref[0] + b_ref[...], 0.0)


def _finale(acc2, y5, bias):
    nb = 10
    bs = N_NODES // nb
    return pl.pallas_call(
        _fin_body,
        grid=(nb,),
        in_specs=[
            pl.BlockSpec((2, bs, OUT_F), lambda i: (0, i, 0)),
            pl.BlockSpec((1, bs, OUT_F), lambda i: (N_REL, i, 0)),
            pl.BlockSpec((1, OUT_F), lambda i: (0, 0)),
        ],
        out_specs=pl.BlockSpec((bs, OUT_F), lambda i: (i, 0)),
        out_shape=jax.ShapeDtypeStruct((N_NODES, OUT_F), jnp.float32),
    )(acc2, y5, bias)


def _sc_deg_kernel(dst, et):
    mesh = plsc.VectorSubcoreMesh(core_axis_name="c", subcore_axis_name="s")
    esub = N_EDGES // NS
    nch_a = esub // CH
    nsup_a = esub // SUP
    deg_slice = _DEG_PAD // NS

    @functools.partial(
        pl.kernel,
        out_type=jax.ShapeDtypeStruct((NC * _DEG_PAD,), jnp.float32),
        mesh=mesh,
        scratch_types=[
            pltpu.VMEM_SHARED((_DEG_PAD,), jnp.float32),
            pltpu.VMEM((2 * SUP,), jnp.int32),
            pltpu.VMEM((2 * SUP,), jnp.int32),
            pltpu.VMEM((NSLOT, CH), jnp.int32),
            pltpu.VMEM((CH,), jnp.float32),
            pltpu.VMEM((deg_slice,), jnp.float32),
            pltpu.SemaphoreType.DMA((NSLOT,)),
            pltpu.SemaphoreType.DMA,
        ],
        compiler_params=pltpu.CompilerParams(needs_layout_passes=False),
    )
    def k(dst_hbm, et_hbm, winv_hbm,
          deg_sh, t_big, d_big, cs_st, ones_st, wsl_v, sem_s, sem_e):
        cid = lax.axis_index("c")
        tid = lax.axis_index("s")
        zeros16 = jnp.zeros((16,), jnp.float32)
        ones16 = jnp.ones((16,), jnp.float32)

        def esup_start(base, eb):
            o = eb * SUP
            pltpu.async_copy(et_hbm.at[pl.ds(base, SUP)],
                             t_big.at[pl.ds(o, SUP)], sem_e)
            pltpu.async_copy(dst_hbm.at[pl.ds(base, SUP)],
                             d_big.at[pl.ds(o, SUP)], sem_e)

        def esup_wait(base, eb):
            o = eb * SUP
            pltpu.make_async_copy(et_hbm.at[pl.ds(base, SUP)],
                                  t_big.at[pl.ds(o, SUP)], sem_e).wait()
            pltpu.make_async_copy(dst_hbm.at[pl.ds(base, SUP)],
                                  d_big.at[pl.ds(o, SUP)], sem_e).wait()

        for kk in range(CH // 16):
            ones_st[pl.ds(kk * 16, 16)] = ones16

        @pl.loop(0, deg_slice // 16)
        def _zw(i):
            wsl_v[pl.ds(i * 16, 16)] = zeros16

        pltpu.sync_copy(wsl_v, deg_sh.at[pl.ds(tid * deg_slice, deg_slice)])
        plsc.subcore_barrier()

        def deg_scat_wait(b):
            pltpu.make_async_copy(
                ones_st, deg_sh.at[cs_st.at[b]], sem_s.at[b]).wait()

        esup_start(tid * esub, 0)

        @pl.loop(0, nch_a)
        def _deg(c):
            k5 = c // 5
            eb = lax.rem(k5, 2)
            b = lax.rem(c, NSLOT)

            @pl.when(lax.rem(c, 5) == 0)
            def _():
                esup_wait(tid * esub + k5 * SUP, eb)

                @pl.when(k5 < nsup_a - 1)
                def _():
                    esup_start(tid * esub + (k5 + 1) * SUP, lax.rem(k5 + 1, 2))

            @pl.when(c >= NSLOT)
            def _():
                deg_scat_wait(b)

            off = eb * SUP + lax.rem(c, 5) * CH
            for kk in range(CH // 16):
                sle = pl.ds(off + kk * 16, 16)
                slb = pl.ds(kk * 16, 16)
                cs_st[b, slb] = t_big[sle] * N_NODES + d_big[sle]
            pltpu.async_copy(ones_st, deg_sh.at[cs_st.at[b]],
                             sem_s.at[b], add=True)

        for b in range(NSLOT):
            deg_scat_wait(b)
        plsc.subcore_barrier()

        pltpu.sync_copy(deg_sh.at[pl.ds(tid * deg_slice, deg_slice)], wsl_v)

        @pl.loop(0, deg_slice // 16)
        def _winv(i):
            sl = pl.ds(i * 16, 16)
            wsl_v[sl] = 1.0 / jnp.maximum(wsl_v[sl], 1.0)

        pltpu.sync_copy(
            wsl_v,
            winv_hbm.at[pl.ds(cid * _DEG_PAD + tid * deg_slice, deg_slice)])

    return k(dst, et)


def _sc_edge_kernel(yflat, src, dst, et, winv):
    mesh = plsc.VectorSubcoreMesh(core_axis_name="c", subcore_axis_name="s")
    ept = N_EDGES // (NC * NS)
    nch_c = ept // CH
    nsup_c = ept // SUP
    deg_slice = _DEG_PAD // NS

    @functools.partial(
        pl.kernel,
        out_type=jax.ShapeDtypeStruct((NC, N_NODES, OUT_F), jnp.float32),
        mesh=mesh,
        scratch_types=[
            pltpu.VMEM_SHARED((N_NODES, OUT_F), jnp.float32),
            pltpu.VMEM_SHARED((_DEG_PAD,), jnp.float32),
            pltpu.VMEM((NSLOT * CH, OUT_F), jnp.float32),
            pltpu.VMEM((2 * SUP,), jnp.int32),
            pltpu.VMEM((2 * SUP,), jnp.int32),
            pltpu.VMEM((2 * SUP,), jnp.int32),
            pltpu.VMEM((NSLOT, CH), jnp.int32),
            pltpu.VMEM((NSLOT, CH), jnp.int32),
            pltpu.VMEM((NSLOT, CH), jnp.int32),
            pltpu.VMEM((NSLOT * CH,), jnp.float32),
            pltpu.VMEM((deg_slice,), jnp.float32),
            pltpu.SemaphoreType.DMA((NSLOT,)),
            pltpu.SemaphoreType.DMA((NSLOT,)),
            pltpu.SemaphoreType.DMA((NSLOT,)),
            pltpu.SemaphoreType.DMA,
        ],
        compiler_params=pltpu.CompilerParams(needs_layout_passes=False),
    )
    def k(y_hbm, src_hbm, dst_hbm, et_hbm, winv_hbm, out_hbm,
          acc_sh, winv_sh, rows2, s_big, t_big, d_big,
          cs_st, id_st, d_ix, w_st, wsl_v,
          sem_r, sem_w, sem_s, sem_e):
        cid = lax.axis_index("c")
        tid = lax.axis_index("s")
        wid = cid * NS + tid
        zeros16 = jnp.zeros((16,), jnp.float32)
        i16 = lambda v: jnp.broadcast_to(v, (16,)).astype(jnp.int32)

        def esup_start(base, eb):
            o = eb * SUP
            pltpu.async_copy(src_hbm.at[pl.ds(base, SUP)],
                             s_big.at[pl.ds(o, SUP)], sem_e)
            pltpu.async_copy(et_hbm.at[pl.ds(base, SUP)],
                             t_big.at[pl.ds(o, SUP)], sem_e)
            pltpu.async_copy(dst_hbm.at[pl.ds(base, SUP)],
                             d_big.at[pl.ds(o, SUP)], sem_e)

        def esup_wait(base, eb):
            o = eb * SUP
            pltpu.make_async_copy(src_hbm.at[pl.ds(base, SUP)],
                                  s_big.at[pl.ds(o, SUP)], sem_e).wait()
            pltpu.make_async_copy(et_hbm.at[pl.ds(base, SUP)],
                                  t_big.at[pl.ds(o, SUP)], sem_e).wait()
            pltpu.make_async_copy(dst_hbm.at[pl.ds(base, SUP)],
                                  d_big.at[pl.ds(o, SUP)], sem_e).wait()

        @pl.loop(0, CH)
        def _zr(e):
            for j in range(OUT_F // 16):
                rows2[e, pl.ds(j * 16, 16)] = zeros16

        pltpu.sync_copy(
            winv_hbm.at[pl.ds(cid * _DEG_PAD + tid * deg_slice, deg_slice)],
            wsl_v)
        pltpu.sync_copy(wsl_v, winv_sh.at[pl.ds(tid * deg_slice, deg_slice)])

        @pl.loop(0, 8)
        def _zacc(kk):
            off = tid * 640 + kk * CH

            @pl.when(off < N_NODES)
            def _():
                pltpu.sync_copy(rows2.at[pl.ds(0, CH)],
                                acc_sh.at[pl.ds(off, CH)])

        plsc.subcore_barrier()

        def prep(c):
            b = lax.rem(c, NSLOT)
            eb = lax.rem(c // 5, 2)
            off = eb * SUP + lax.rem(c, 5) * CH
            for kk in range(CH // 16):
                sle = pl.ds(off + kk * 16, 16)
                slb = pl.ds(kk * 16, 16)
                t16 = t_big[sle]
                cs_st[b, slb] = t16 * N_NODES + s_big[sle]
                id_st[b, slb] = t16 * N_NODES + d_big[sle]
                d_ix[b, slb] = d_big[sle]
            pltpu.async_copy(y_hbm.at[cs_st.at[b]],
                             rows2.at[pl.ds(b * CH, CH)], sem_r.at[b])
            pltpu.async_copy(winv_sh.at[id_st.at[b]],
                             w_st.at[pl.ds(b * CH, CH)], sem_w.at[b])

        def process(c):
            b = lax.rem(c, NSLOT)
            pltpu.make_async_copy(y_hbm.at[cs_st.at[b]],
                                  rows2.at[pl.ds(b * CH, CH)],
                                  sem_r.at[b]).wait()
            pltpu.make_async_copy(winv_sh.at[id_st.at[b]],
                                  w_st.at[pl.ds(b * CH, CH)],
                                  sem_w.at[b]).wait()

            @plsc.parallel_loop(0, CH, unroll=8)
            def _scale(e):
                rb = b * CH + e
                wspl = plsc.load_gather(w_st, [i16(rb)])
                for j in range(OUT_F // 16):
                    sl2 = pl.ds(j * 16, 16)
                    rows2[rb, sl2] = rows2[rb, sl2] * wspl

            pltpu.async_copy(rows2.at[pl.ds(b * CH, CH)],
                             acc_sh.at[d_ix.at[b]], sem_s.at[b], add=True)

        def scat_wait(b):
            pltpu.make_async_copy(rows2.at[pl.ds(b * CH, CH)],
                                  acc_sh.at[d_ix.at[b]], sem_s.at[b]).wait()

        cbase = wid * ept
        esup_start(cbase, 0)
        esup_wait(cbase, 0)
        esup_start(cbase + SUP, 1)
        prep(jnp.int32(0))
        prep(jnp.int32(1))
        prep(jnp.int32(2))
        prep(jnp.int32(3))
        process(jnp.int32(0))
        process(jnp.int32(1))

        @pl.loop(NSLOT, nch_c)
        def _mc(c):
            k5 = c // 5

            @pl.when(lax.rem(c, 5) == 0)
            def _():
                esup_wait(cbase + k5 * SUP, lax.rem(k5, 2))

                @pl.when(k5 < nsup_c - 1)
                def _():
                    esup_start(cbase + (k5 + 1) * SUP, lax.rem(k5 + 1, 2))

            scat_wait(lax.rem(c, NSLOT))
            prep(c)
            process(c - 2)

        process(jnp.int32(nch_c - 2))
        process(jnp.int32(nch_c - 1))
        for b in range(NSLOT):
            scat_wait(jnp.int32(b))
        plsc.subcore_barrier()

        @pl.loop(0, 8)
        def _wb(kk):
            off = tid * 640 + kk * CH

            @pl.when(off < N_NODES)
            def _():
                pltpu.sync_copy(acc_sh.at[pl.ds(off, CH)],
                                rows2.at[pl.ds(0, CH)])
                pltpu.sync_copy(rows2.at[pl.ds(0, CH)],
                                out_hbm.at[cid, pl.ds(off, CH)])

    return k(yflat, src, dst, et, winv)


def kernel(x, edge_index, edge_type, weight, loop_weight, h_bias):
    src = edge_index[0].astype(jnp.int32)
    dst = edge_index[1].astype(jnp.int32)
    et = edge_type.astype(jnp.int32)
    w5 = jnp.concatenate([weight, loop_weight[None]], axis=0)
    winv = _sc_deg_kernel(dst, et)
    y5 = _relation_matmul(x, w5)
    yflat = y5.reshape((N_REL + 1) * N_NODES, OUT_F)
    acc2 = _sc_edge_kernel(yflat, src, dst, et, winv)
    return _finale(acc2, y5, h_bias.reshape(1, OUT_F))


# speedup vs baseline: 1.0076x; 1.0076x over previous
import functools

import jax
import jax.numpy as jnp
from jax import lax
from jax.experimental import pallas as pl
from jax.experimental.pallas import tpu as pltpu
from jax.experimental.pallas import tpu_sc as plsc

N_NODES = 10000
IN_F = 128
OUT_F = 128
N_REL = 4
N_EDGES = 320000

NC = 2
NS = 16
CH = 80
NSLOT = 4
SUP = 5 * CH
_DEG_PAD = 40960


def _mm_body(x_ref, w_ref, y_ref):
    x = x_ref[...]
    for r in range(N_REL + 1):
        y_ref[r] = jnp.dot(x, w_ref[r], preferred_element_type=jnp.float32)


def _relation_matmul(x, w5):
    nb = 10
    bs = N_NODES // nb
    r = w5.shape[0]
    return pl.pallas_call(
        _mm_body,
        grid=(nb,),
        in_specs=[
            pl.BlockSpec((bs, IN_F), lambda i: (i, 0)),
            pl.BlockSpec((r, IN_F, OUT_F), lambda i: (0, 0, 0)),
        ],
        out_specs=pl.BlockSpec((r, bs, OUT_F), lambda i: (0, i, 0)),
        out_shape=jax.ShapeDtypeStruct((r, N_NODES, OUT_F), jnp.float32),
    )(x, w5)


def _fin_body(a_ref, y5_ref, b_ref, o_ref):
    o_ref[...] = jnp.maximum(
        a_ref[0] + a_ref[1] + y5_ref[0] + b_ref[...], 0.0)


def _finale(acc2, y5, bias):
    nb = 10
    bs = N_NODES // nb
    return pl.pallas_call(
        _fin_body,
        grid=(nb,),
        in_specs=[
            pl.BlockSpec((2, bs, OUT_F), lambda i: (0, i, 0)),
            pl.BlockSpec((1, bs, OUT_F), lambda i: (N_REL, i, 0)),
            pl.BlockSpec((1, OUT_F), lambda i: (0, 0)),
        ],
        out_specs=pl.BlockSpec((bs, OUT_F), lambda i: (i, 0)),
        out_shape=jax.ShapeDtypeStruct((N_NODES, OUT_F), jnp.float32),
    )(acc2, y5, bias)


def _sc_deg_kernel(dst, et):
    mesh = plsc.VectorSubcoreMesh(core_axis_name="c", subcore_axis_name="s")
    esub = N_EDGES // NS
    nch_a = esub // CH
    nsup_a = esub // SUP
    deg_slice = _DEG_PAD // NS

    @functools.partial(
        pl.kernel,
        out_type=jax.ShapeDtypeStruct((NC * _DEG_PAD,), jnp.float32),
        mesh=mesh,
        scratch_types=[
            pltpu.VMEM_SHARED((_DEG_PAD,), jnp.float32),
            pltpu.VMEM((2 * SUP,), jnp.int32),
            pltpu.VMEM((2 * SUP,), jnp.int32),
            pltpu.VMEM((NSLOT, CH), jnp.int32),
            pltpu.VMEM((CH,), jnp.float32),
            pltpu.VMEM((deg_slice,), jnp.float32),
            pltpu.SemaphoreType.DMA((NSLOT,)),
            pltpu.SemaphoreType.DMA,
        ],
        compiler_params=pltpu.CompilerParams(needs_layout_passes=False),
    )
    def k(dst_hbm, et_hbm, winv_hbm,
          deg_sh, t_big, d_big, cs_st, ones_st, wsl_v, sem_s, sem_e):
        cid = lax.axis_index("c")
        tid = lax.axis_index("s")
        zeros16 = jnp.zeros((16,), jnp.float32)
        ones16 = jnp.ones((16,), jnp.float32)

        def esup_start(base, eb):
            o = eb * SUP
            pltpu.async_copy(et_hbm.at[pl.ds(base, SUP)],
                             t_big.at[pl.ds(o, SUP)], sem_e)
            pltpu.async_copy(dst_hbm.at[pl.ds(base, SUP)],
                             d_big.at[pl.ds(o, SUP)], sem_e)

        def esup_wait(base, eb):
            o = eb * SUP
            pltpu.make_async_copy(et_hbm.at[pl.ds(base, SUP)],
                                  t_big.at[pl.ds(o, SUP)], sem_e).wait()
            pltpu.make_async_copy(dst_hbm.at[pl.ds(base, SUP)],
                                  d_big.at[pl.ds(o, SUP)], sem_e).wait()

        for kk in range(CH // 16):
            ones_st[pl.ds(kk * 16, 16)] = ones16

        @pl.loop(0, deg_slice // 16)
        def _zw(i):
            wsl_v[pl.ds(i * 16, 16)] = zeros16

        pltpu.sync_copy(wsl_v, deg_sh.at[pl.ds(tid * deg_slice, deg_slice)])
        plsc.subcore_barrier()

        def deg_scat_wait(b):
            pltpu.make_async_copy(
                ones_st, deg_sh.at[cs_st.at[b]], sem_s.at[b]).wait()

        esup_start(tid * esub, 0)

        @pl.loop(0, nch_a)
        def _deg(c):
            k5 = c // 5
            eb = lax.rem(k5, 2)
            b = lax.rem(c, NSLOT)

            @pl.when(lax.rem(c, 5) == 0)
            def _():
                esup_wait(tid * esub + k5 * SUP, eb)

                @pl.when(k5 < nsup_a - 1)
                def _():
                    esup_start(tid * esub + (k5 + 1) * SUP, lax.rem(k5 + 1, 2))

            @pl.when(c >= NSLOT)
            def _():
                deg_scat_wait(b)

            off = eb * SUP + lax.rem(c, 5) * CH
            for kk in range(CH // 16):
                sle = pl.ds(off + kk * 16, 16)
                slb = pl.ds(kk * 16, 16)
                cs_st[b, slb] = t_big[sle] * N_NODES + d_big[sle]
            pltpu.async_copy(ones_st, deg_sh.at[cs_st.at[b]],
                             sem_s.at[b], add=True)

        for b in range(NSLOT):
            deg_scat_wait(b)
        plsc.subcore_barrier()

        pltpu.sync_copy(deg_sh.at[pl.ds(tid * deg_slice, deg_slice)], wsl_v)

        @pl.loop(0, deg_slice // 16)
        def _winv(i):
            sl = pl.ds(i * 16, 16)
            wsl_v[sl] = 1.0 / jnp.maximum(wsl_v[sl], 1.0)

        pltpu.sync_copy(
            wsl_v,
            winv_hbm.at[pl.ds(cid * _DEG_PAD + tid * deg_slice, deg_slice)])

    return k(dst, et)


def _sc_edge_kernel(yflat, src, dst, et, winv):
    mesh = plsc.VectorSubcoreMesh(core_axis_name="c", subcore_axis_name="s")
    ept = N_EDGES // (NC * NS)
    nch_c = ept // CH
    nsup_c = ept // SUP
    deg_slice = _DEG_PAD // NS

    @functools.partial(
        pl.kernel,
        out_type=jax.ShapeDtypeStruct((NC, N_NODES, OUT_F), jnp.float32),
        mesh=mesh,
        scratch_types=[
            pltpu.VMEM_SHARED((N_NODES, OUT_F), jnp.float32),
            pltpu.VMEM_SHARED((_DEG_PAD,), jnp.float32),
            pltpu.VMEM((NSLOT * CH, OUT_F), jnp.float32),
            pltpu.VMEM((2 * SUP,), jnp.int32),
            pltpu.VMEM((2 * SUP,), jnp.int32),
            pltpu.VMEM((2 * SUP,), jnp.int32),
            pltpu.VMEM((NSLOT, CH), jnp.int32),
            pltpu.VMEM((NSLOT, CH), jnp.int32),
            pltpu.VMEM((NSLOT, CH), jnp.int32),
            pltpu.VMEM((NSLOT * CH,), jnp.float32),
            pltpu.VMEM((deg_slice,), jnp.float32),
            pltpu.SemaphoreType.DMA((NSLOT,)),
            pltpu.SemaphoreType.DMA((NSLOT,)),
            pltpu.SemaphoreType.DMA((NSLOT,)),
            pltpu.SemaphoreType.DMA,
        ],
        compiler_params=pltpu.CompilerParams(needs_layout_passes=False),
    )
    def k(y_hbm, src_hbm, dst_hbm, et_hbm, winv_hbm, out_hbm,
          acc_sh, winv_sh, rows2, s_big, t_big, d_big,
          cs_st, id_st, d_ix, w_st, wsl_v,
          sem_r, sem_w, sem_s, sem_e):
        cid = lax.axis_index("c")
        tid = lax.axis_index("s")
        wid = cid * NS + tid
        zeros16 = jnp.zeros((16,), jnp.float32)
        i16 = lambda v: jnp.broadcast_to(v, (16,)).astype(jnp.int32)

        def esup_start(base, eb):
            o = eb * SUP
            pltpu.async_copy(src_hbm.at[pl.ds(base, SUP)],
                             s_big.at[pl.ds(o, SUP)], sem_e)
            pltpu.async_copy(et_hbm.at[pl.ds(base, SUP)],
                             t_big.at[pl.ds(o, SUP)], sem_e)
            pltpu.async_copy(dst_hbm.at[pl.ds(base, SUP)],
                             d_big.at[pl.ds(o, SUP)], sem_e)

        def esup_wait(base, eb):
            o = eb * SUP
            pltpu.make_async_copy(src_hbm.at[pl.ds(base, SUP)],
                                  s_big.at[pl.ds(o, SUP)], sem_e).wait()
            pltpu.make_async_copy(et_hbm.at[pl.ds(base, SUP)],
                                  t_big.at[pl.ds(o, SUP)], sem_e).wait()
            pltpu.make_async_copy(dst_hbm.at[pl.ds(base, SUP)],
                                  d_big.at[pl.ds(o, SUP)], sem_e).wait()

        @pl.loop(0, CH)
        def _zr(e):
            for j in range(OUT_F // 16):
                rows2[e, pl.ds(j * 16, 16)] = zeros16

        pltpu.sync_copy(
            winv_hbm.at[pl.ds(cid * _DEG_PAD + tid * deg_slice, deg_slice)],
            wsl_v)
        pltpu.sync_copy(wsl_v, winv_sh.at[pl.ds(tid * deg_slice, deg_slice)])

        @pl.loop(0, 8)
        def _zacc(kk):
            off = tid * 640 + kk * CH

            @pl.when(off < N_NODES)
            def _():
                pltpu.sync_copy(rows2.at[pl.ds(0, CH)],
                                acc_sh.at[pl.ds(off, CH)])

        plsc.subcore_barrier()

        def prep(c):
            b = lax.rem(c, NSLOT)
            eb = lax.rem(c // 5, 2)
            off = eb * SUP + lax.rem(c, 5) * CH
            for kk in range(CH // 16):
                sle = pl.ds(off + kk * 16, 16)
                slb = pl.ds(kk * 16, 16)
                t16 = t_big[sle]
                cs_st[b, slb] = t16 * N_NODES + s_big[sle]
                id_st[b, slb] = t16 * N_NODES + d_big[sle]
                d_ix[b, slb] = d_big[sle]
            pltpu.async_copy(y_hbm.at[cs_st.at[b]],
                             rows2.at[pl.ds(b * CH, CH)], sem_r.at[b])
            pltpu.async_copy(winv_sh.at[id_st.at[b]],
                             w_st.at[pl.ds(b * CH, CH)], sem_w.at[b])

        def process(c):
            b = lax.rem(c, NSLOT)
            pltpu.make_async_copy(y_hbm.at[cs_st.at[b]],
                                  rows2.at[pl.ds(b * CH, CH)],
                                  sem_r.at[b]).wait()
            pltpu.make_async_copy(winv_sh.at[id_st.at[b]],
                                  w_st.at[pl.ds(b * CH, CH)],
                                  sem_w.at[b]).wait()

            @plsc.parallel_loop(0, CH, unroll=4)
            def _scale(e):
                rb = b * CH + e
                wspl = plsc.load_gather(w_st, [i16(rb)])
                for j in range(OUT_F // 16):
                    sl2 = pl.ds(j * 16, 16)
                    rows2[rb, sl2] = rows2[rb, sl2] * wspl

            pltpu.async_copy(rows2.at[pl.ds(b * CH, CH)],
                             acc_sh.at[d_ix.at[b]], sem_s.at[b], add=True)

        def scat_wait(b):
            pltpu.make_async_copy(rows2.at[pl.ds(b * CH, CH)],
                                  acc_sh.at[d_ix.at[b]], sem_s.at[b]).wait()

        cbase = wid * ept
        esup_start(cbase, 0)
        esup_wait(cbase, 0)
        esup_start(cbase + SUP, 1)
        prep(jnp.int32(0))
        prep(jnp.int32(1))
        prep(jnp.int32(2))
        prep(jnp.int32(3))
        process(jnp.int32(0))
        process(jnp.int32(1))

        @pl.loop(NSLOT, nch_c)
        def _mc(c):
            k5 = c // 5

            @pl.when(lax.rem(c, 5) == 0)
            def _():
                esup_wait(cbase + k5 * SUP, lax.rem(k5, 2))

                @pl.when(k5 < nsup_c - 1)
                def _():
                    esup_start(cbase + (k5 + 1) * SUP, lax.rem(k5 + 1, 2))

            scat_wait(lax.rem(c, NSLOT))
            prep(c)
            process(c - 2)

        process(jnp.int32(nch_c - 2))
        process(jnp.int32(nch_c - 1))
        for b in range(NSLOT):
            scat_wait(jnp.int32(b))
        plsc.subcore_barrier()

        @pl.loop(0, 8)
        def _wb(kk):
            off = tid * 640 + kk * CH

            @pl.when(off < N_NODES)
            def _():
                pltpu.sync_copy(acc_sh.at[pl.ds(off, CH)],
                                rows2.at[pl.ds(0, CH)])
                pltpu.sync_copy(rows2.at[pl.ds(0, CH)],
                                out_hbm.at[cid, pl.ds(off, CH)])

    return k(yflat, src, dst, et, winv)


def kernel(x, edge_index, edge_type, weight, loop_weight, h_bias):
    src = edge_index[0].astype(jnp.int32)
    dst = edge_index[1].astype(jnp.int32)
    et = edge_type.astype(jnp.int32)
    w5 = jnp.concatenate([weight, loop_weight[None]], axis=0)
    winv = _sc_deg_kernel(dst, et)
    y5 = _relation_matmul(x, w5)
    yflat = y5.reshape((N_REL + 1) * N_NODES, OUT_F)
    acc2 = _sc_edge_kernel(yflat, src, dst, et, winv)
    return _finale(acc2, y5, h_bias.reshape(1, OUT_F))
